# Initial kernel scaffold; baseline (speedup 1.0000x reference)
import jax
import jax.numpy as jnp
from jax.experimental import pallas as pl


def kernel(x, edge_index, batch, W1a, b1a, g1, be1, W1b, b1b, W2a, b2a, g2, be2, W2b, b2b, W3a, b3a, g3, be3, W3b, b3b, Wl1, bl1, Wl2, bl2):
    raise NotImplementedError("write your pallas kernel here")


# speedup vs baseline: 5.5276x; 5.5276x over previous
import functools

import jax
import jax.numpy as jnp
from jax import lax
from jax.experimental import pallas as pl
from jax.experimental.pallas import tpu as pltpu
from jax.experimental.pallas import tpu_sc as plsc

N = 10000
E = 320000
HID = 256
G = 64

NTILES = 16
CHUNK = 128
GRP = 8
CHUNKS = 160
CHUNKS32 = 80
E_PAD = NTILES * CHUNKS * CHUNK
ACC_ROWS = 10240
RPT = ACC_ROWS // NTILES

BM = 1000
NBLK = N // BM


_MESH = plsc.VectorSubcoreMesh(core_axis_name="c", subcore_axis_name="s")


def _zero_acc(rows, acc, s):
  def zrow(i, carry):
    for j in range(128 // 16):
      rows[i, pl.ds(j * 16, 16)] = jnp.zeros((16,), jnp.float32)
    return carry

  lax.fori_loop(0, CHUNK, zrow, 0)
  base = s * RPT
  for k in range(RPT // CHUNK):
    pltpu.sync_copy(rows, acc.at[pl.ds(base + k * CHUNK, CHUNK)])


@functools.partial(
    pl.kernel,
    mesh=_MESH,
    out_type=[
        jax.ShapeDtypeStruct((ACC_ROWS, 128), jnp.float32),
        jax.ShapeDtypeStruct((ACC_ROWS, 128), jnp.float32),
    ],
    scratch_types=[
        pltpu.VMEM((GRP, CHUNK), jnp.int32),
        pltpu.VMEM((GRP, CHUNK), jnp.int32),
        pltpu.VMEM((CHUNK, 128), jnp.float32),
        pltpu.VMEM_SHARED((ACC_ROWS, 128), jnp.float32),
    ],
)
def _agg_edge_split(x_hbm, src_hbm, dst_hbm, out0, out1,
                    srcv, dstv, rows, acc):
  c = lax.axis_index("c")
  s = lax.axis_index("s")
  wid = c * NTILES + s

  _zero_acc(rows, acc, s)
  plsc.subcore_barrier()

  def group(g, carry):
    pltpu.sync_copy(src_hbm.at[wid, pl.ds(g * GRP, GRP)], srcv)
    pltpu.sync_copy(dst_hbm.at[wid, pl.ds(g * GRP, GRP)], dstv)
    for j in range(GRP):
      pltpu.sync_copy(x_hbm.at[srcv.at[j]], rows)
      pltpu.sync_copy(rows, acc.at[dstv.at[j]], add=True)
    return carry

  lax.fori_loop(0, CHUNKS32 // GRP, group, 0)

  plsc.subcore_barrier()
  base = s * RPT

  @pl.when(c == 0)
  def _():
    pltpu.sync_copy(acc.at[pl.ds(base, RPT)], out0.at[pl.ds(base, RPT)])

  @pl.when(c == 1)
  def _():
    pltpu.sync_copy(acc.at[pl.ds(base, RPT)], out1.at[pl.ds(base, RPT)])


@functools.partial(
    pl.kernel,
    mesh=_MESH,
    out_type=[
        jax.ShapeDtypeStruct((ACC_ROWS, 128), jnp.float32),
        jax.ShapeDtypeStruct((ACC_ROWS, 128), jnp.float32),
    ],
    scratch_types=[
        pltpu.VMEM((GRP, CHUNK), jnp.int32),
        pltpu.VMEM((GRP, CHUNK), jnp.int32),
        pltpu.VMEM((CHUNK, 128), jnp.float32),
        pltpu.VMEM_SHARED((ACC_ROWS, 128), jnp.float32),
    ],
)
def _agg_feat_split(x0_hbm, x1_hbm, src_hbm, dst_hbm, out0, out1,
                    srcv, dstv, rows, acc):
  c = lax.axis_index("c")
  s = lax.axis_index("s")

  def run(x_t, out_t):
    _zero_acc(rows, acc, s)
    plsc.subcore_barrier()

    def group(g, carry):
      pltpu.sync_copy(src_hbm.at[s, pl.ds(g * GRP, GRP)], srcv)
      pltpu.sync_copy(dst_hbm.at[s, pl.ds(g * GRP, GRP)], dstv)
      for j in range(GRP):
        pltpu.sync_copy(x_t.at[srcv.at[j]], rows)
        pltpu.sync_copy(rows, acc.at[dstv.at[j]], add=True)
      return carry

    lax.fori_loop(0, CHUNKS // GRP, group, 0)

    plsc.subcore_barrier()
    base = s * RPT
    pltpu.sync_copy(acc.at[pl.ds(base, RPT)], out_t.at[pl.ds(base, RPT)])

  @pl.when(c == 0)
  def _():
    run(x0_hbm, out0)

  @pl.when(c == 1)
  def _():
    run(x1_hbm, out1)


def _mlp_a_common(i, sfull, war, bar, h1r, statr):
  h1 = lax.dot_general(sfull, war[...], (((1,), (1,)), ((), ())),
                       preferred_element_type=jnp.float32) + bar[...]
  h1r[...] = h1

  @pl.when(i == 0)
  def _():
    statr[...] = jnp.zeros_like(statr)

  part = jnp.concatenate(
      [h1.sum(axis=0, keepdims=True),
       (h1 * h1).sum(axis=0, keepdims=True),
       jnp.zeros((6, HID), jnp.float32)], axis=0)
  statr[...] += part


def _mlp_a1_body(xr, a0r, a1r, war, bar, h1r, statr):
  _mlp_a_common(pl.program_id(0), xr[...] + a0r[...] + a1r[...],
                war, bar, h1r, statr)


_mlp_a1 = pl.pallas_call(
    _mlp_a1_body,
    grid=(NBLK,),
    in_specs=[
        pl.BlockSpec((BM, 128), lambda i: (i, 0)),
        pl.BlockSpec((BM, 128), lambda i: (i, 0)),
        pl.BlockSpec((BM, 128), lambda i: (i, 0)),
        pl.BlockSpec((HID, 128), lambda i: (0, 0)),
        pl.BlockSpec((1, HID), lambda i: (0, 0)),
    ],
    out_specs=[
        pl.BlockSpec((BM, HID), lambda i: (i, 0)),
        pl.BlockSpec((8, HID), lambda i: (0, 0)),
    ],
    out_shape=[
        jax.ShapeDtypeStruct((N, HID), jnp.float32),
        jax.ShapeDtypeStruct((8, HID), jnp.float32),
    ],
)


def _mlp_a2_body(x0r, x1r, a0r, a1r, war, bar, h1r, statr):
  sfull = jnp.concatenate([x0r[...] + a0r[...], x1r[...] + a1r[...]], axis=1)
  _mlp_a_common(pl.program_id(0), sfull, war, bar, h1r, statr)


_mlp_a2 = pl.pallas_call(
    _mlp_a2_body,
    grid=(NBLK,),
    in_specs=[
        pl.BlockSpec((BM, 128), lambda i: (i, 0)),
        pl.BlockSpec((BM, 128), lambda i: (i, 0)),
        pl.BlockSpec((BM, 128), lambda i: (i, 0)),
        pl.BlockSpec((BM, 128), lambda i: (i, 0)),
        pl.BlockSpec((HID, HID), lambda i: (0, 0)),
        pl.BlockSpec((1, HID), lambda i: (0, 0)),
    ],
    out_specs=[
        pl.BlockSpec((BM, HID), lambda i: (i, 0)),
        pl.BlockSpec((8, HID), lambda i: (0, 0)),
    ],
    out_shape=[
        jax.ShapeDtypeStruct((N, HID), jnp.float32),
        jax.ShapeDtypeStruct((8, HID), jnp.float32),
    ],
)


def _mlp_b_body(h1r, statr, gr, ber, wbr, bbr, o0r, o1r):
  st = statr[...]
  mean = st[0:1, :] * (1.0 / N)
  var = st[1:2, :] * (1.0 / N) - mean * mean
  s1 = gr[...] * lax.rsqrt(var + 1e-5)
  s2 = ber[...] - mean * s1
  y = jnp.maximum(h1r[...] * s1 + s2, 0.0)
  h2 = lax.dot_general(y, wbr[...], (((1,), (1,)), ((), ())),
                       preferred_element_type=jnp.float32) + bbr[...]
  h2 = jnp.maximum(h2, 0.0)
  o0r[...] = h2[:, :HID // 2]
  o1r[...] = h2[:, HID // 2:]


_mlp_b = pl.pallas_call(
    _mlp_b_body,
    grid=(NBLK,),
    in_specs=[
        pl.BlockSpec((BM, HID), lambda i: (i, 0)),
        pl.BlockSpec((8, HID), lambda i: (0, 0)),
        pl.BlockSpec((1, HID), lambda i: (0, 0)),
        pl.BlockSpec((1, HID), lambda i: (0, 0)),
        pl.BlockSpec((HID, HID), lambda i: (0, 0)),
        pl.BlockSpec((1, HID), lambda i: (0, 0)),
    ],
    out_specs=[
        pl.BlockSpec((BM, HID // 2), lambda i: (i, 0)),
        pl.BlockSpec((BM, HID // 2), lambda i: (i, 0)),
    ],
    out_shape=[
        jax.ShapeDtypeStruct((N, HID // 2), jnp.float32),
        jax.ShapeDtypeStruct((N, HID // 2), jnp.float32),
    ],
)


def _pool_body(h0r, h1r, br, wl1r, bl1r, wl2r, bl2r, outr, accr):
  i = pl.program_id(0)

  @pl.when(i == 0)
  def _():
    accr[...] = jnp.zeros_like(accr)

  h = jnp.concatenate([h0r[...], h1r[...]], axis=1)
  b = br[0, 0, :]
  onehot = (lax.broadcasted_iota(jnp.int32, (G, BM), 0)
            == b[None, :]).astype(jnp.float32)
  accr[...] += lax.dot_general(onehot, h, (((1,), (0,)), ((), ())),
                               preferred_element_type=jnp.float32)

  @pl.when(i == NBLK - 1)
  def _():
    p = accr[...]
    y = lax.dot_general(p, wl1r[...], (((1,), (1,)), ((), ())),
                        preferred_element_type=jnp.float32) + bl1r[...]
    y = jnp.maximum(y, 0.0)
    out = lax.dot_general(y, wl2r[...], (((1,), (1,)), ((), ())),
                          preferred_element_type=jnp.float32) + bl2r[...]
    outr[...] = out


_pool = pl.pallas_call(
    _pool_body,
    grid=(NBLK,),
    in_specs=[
        pl.BlockSpec((BM, HID // 2), lambda i: (i, 0)),
        pl.BlockSpec((BM, HID // 2), lambda i: (i, 0)),
        pl.BlockSpec((1, 1, BM), lambda i: (i, 0, 0)),
        pl.BlockSpec((HID, HID), lambda i: (0, 0)),
        pl.BlockSpec((1, HID), lambda i: (0, 0)),
        pl.BlockSpec((128, HID), lambda i: (0, 0)),
        pl.BlockSpec((1, 128), lambda i: (0, 0)),
    ],
    out_specs=pl.BlockSpec((G, 128), lambda i: (0, 0)),
    out_shape=jax.ShapeDtypeStruct((G, 128), jnp.float32),
    scratch_shapes=[pltpu.VMEM((G, HID), jnp.float32)],
)


def kernel(x, edge_index, batch, W1a, b1a, g1, be1, W1b, b1b,
           W2a, b2a, g2, be2, W2b, b2b, W3a, b3a, g3, be3, W3b, b3b,
           Wl1, bl1, Wl2, bl2):
  src = edge_index[0].astype(jnp.int32)
  dst = edge_index[1].astype(jnp.int32)

  def pad_edges(v, n_pad_to, shape, is_dst):
    pad_i = jnp.arange(n_pad_to - E, dtype=jnp.int32)
    fill = (N + pad_i % (ACC_ROWS - N)) if is_dst else (pad_i % 16)
    return jnp.concatenate([v, fill]).reshape(shape)

  srcr16 = pad_edges(src, E_PAD, (NTILES, CHUNKS, CHUNK), False)
  dstr16 = pad_edges(dst, E_PAD, (NTILES, CHUNKS, CHUNK), True)
  srcr32 = srcr16.reshape(32, CHUNKS32, CHUNK)
  dstr32 = dstr16.reshape(32, CHUNKS32, CHUNK)

  def rr(v):
    return v.reshape(1, HID)

  a0, a1 = _agg_edge_split(x, srcr32, dstr32)
  h1, st = _mlp_a1(x, a0, a1, W1a, rr(b1a))
  h0, h1_ = _mlp_b(h1, st, rr(g1), rr(be1), W1b, rr(b1b))

  a0, a1 = _agg_feat_split(h0, h1_, srcr16, dstr16)
  h1, st = _mlp_a2(h0, h1_, a0, a1, W2a, rr(b2a))
  h0, h1_ = _mlp_b(h1, st, rr(g2), rr(be2), W2b, rr(b2b))

  a0, a1 = _agg_feat_split(h0, h1_, srcr16, dstr16)
  h1, st = _mlp_a2(h0, h1_, a0, a1, W3a, rr(b3a))
  h0, h1_ = _mlp_b(h1, st, rr(g3), rr(be3), W3b, rr(b3b))

  batch3 = batch.astype(jnp.int32).reshape(NBLK, 1, BM)
  wl2p = jnp.zeros((128, HID), jnp.float32).at[0].set(Wl2[0])
  bl2p = jnp.zeros((1, 128), jnp.float32).at[0, 0].set(bl2[0])
  out = _pool(h0, h1_, batch3, Wl1, rr(bl1), wl2p, bl2p)
  return out[:, :1]
